# baseline (speedup 1.0000x reference)
import functools

import jax
import jax.numpy as jnp
from jax import lax
from jax.experimental import pallas as pl
from jax.experimental.pallas import tpu as pltpu
from jax.experimental.pallas import tpu_sc as plsc

N = 10000
E = 320000
D = 128
NC = 2
NS = 16
NW = NC * NS
L = 16

EPT = E // NW
K = 80
NCHUNK = EPT // K
RPT = 624
NBR = 3
NBE = 5

_GDN = lax.GatherDimensionNumbers(
    offset_dims=(), collapsed_slice_dims=(0,), start_index_map=(0,))


def _bcast_lane(vec, i):
    idx = jnp.full((L, 1), i, jnp.int32)
    return lax.gather(vec, idx, dimension_numbers=_GDN, slice_sizes=(1,),
                      mode=lax.GatherScatterMode.PROMISE_IN_BOUNDS)


def _sc_body(h, src1, dst1, ew1, p0, p1, acc, src_r, dst_r, ew_r, rows_v,
             se, sg, ss):
    c = lax.axis_index("c")
    s = lax.axis_index("s")
    w = c * NS + s

    zvec = jnp.zeros((L,), jnp.float32)

    def z_body(i, carry):
        for k in range(D // L):
            rows_v[0, i, pl.ds(k * L, L)] = zvec
        return carry

    lax.fori_loop(0, K, z_body, 0, unroll=4)
    rbase = s * RPT
    rem = N - NS * RPT
    zb = rows_v.at[0]
    for j in range(RPT // K):
        pltpu.sync_copy(zb, acc.at[pl.ds(rbase + j * K, K)])
    pltpu.sync_copy(zb.at[pl.ds(0, RPT - (RPT // K) * K)],
                    acc.at[pl.ds(rbase + (RPT // K) * K,
                                 RPT - (RPT // K) * K)])

    @pl.when(s == NS - 1)
    def _():
        pltpu.sync_copy(zb.at[pl.ds(0, rem)],
                        acc.at[pl.ds(NS * RPT, rem)])

    plsc.subcore_barrier()

    def _issue_e(t, me):
        base = w * EPT + t * K
        pltpu.async_copy(src1.at[pl.ds(base, K)], src_r.at[me], se)
        pltpu.async_copy(dst1.at[pl.ds(base, K)], dst_r.at[me], se)
        pltpu.async_copy(ew1.at[pl.ds(base, K)], ew_r.at[me], se)

    def _wait_e(t, me):
        pltpu.make_async_copy(src1.at[pl.ds(0, K)], src_r.at[me], se).wait()
        pltpu.make_async_copy(dst1.at[pl.ds(0, K)], dst_r.at[me], se).wait()
        pltpu.make_async_copy(ew1.at[pl.ds(0, K)], ew_r.at[me], se).wait()

    def _issue_g(t, me, m):
        pltpu.async_copy(h.at[src_r.at[me]], rows_v.at[m], sg)

    def _wait_g(t, me, m):
        pltpu.make_async_copy(h.at[src_r.at[me]], rows_v.at[m], sg).wait()

    def _issue_s(t, me, m):
        pltpu.async_copy(rows_v.at[m], acc.at[dst_r.at[me]], ss, add=True)

    def _wait_s(t, me, m):
        pltpu.make_async_copy(rows_v.at[m], acc.at[dst_r.at[me]],
                              ss).wait()

    def _scale(me, m):
        def grp_body(g, carry2):
            ewv = ew_r[me, pl.ds(g * L, L)]
            r0 = g * L
            for i in range(L):
                wv = _bcast_lane(ewv, i)
                for k in range(D // L):
                    sl = pl.ds(k * L, L)
                    rows_v[m, r0 + i, sl] = rows_v[m, r0 + i, sl] * wv
            return carry2

        lax.fori_loop(0, K // L, grp_body, 0)

    def body(t, m, me, skip_swait=False, skip_gissue=False,
             skip_eissue=False):
        if not skip_swait:
            _wait_s(t - 2, lax.rem(t - 2, NBE) if not isinstance(t, int)
                    else (t - 2) % NBE,
                    lax.rem(t - 2, NBR) if not isinstance(t, int)
                    else (t - 2) % NBR)
        if not skip_gissue:
            me1 = lax.rem(t + 1, NBE) if not isinstance(t, int) \
                else (t + 1) % NBE
            m1 = lax.rem(t + 1, NBR) if not isinstance(t, int) \
                else (t + 1) % NBR
            _wait_e(t + 1, me1)
            _issue_g(t + 1, me1, m1)
        if not skip_eissue:
            me3 = lax.rem(t + 3, NBE) if not isinstance(t, int) \
                else (t + 3) % NBE
            _issue_e(t + 3, me3)
        _wait_g(t, me, m)
        _scale(me, m)
        _issue_s(t, me, m)

    _issue_e(0, 0)
    _issue_e(1, 1)
    _issue_e(2, 2)
    _wait_e(0, 0)
    _issue_g(0, 0, 0)

    body(0, 0, 0, skip_swait=True)
    body(1, 1, 1, skip_swait=True)

    def loop_body(t, carry):
        body(t, lax.rem(t, NBR), lax.rem(t, NBE))
        return carry

    lax.fori_loop(2, NCHUNK - 3, loop_body, 0)

    t = NCHUNK - 3
    body(t, t % NBR, t % NBE, skip_eissue=True)
    t = NCHUNK - 2
    body(t, t % NBR, t % NBE, skip_eissue=True)
    t = NCHUNK - 1
    body(t, t % NBR, t % NBE, skip_eissue=True, skip_gissue=True)

    for t in (NCHUNK - 2, NCHUNK - 1):
        _wait_s(t, t % NBE, t % NBR)
    plsc.subcore_barrier()

    @pl.when(c == 0)
    def _():
        pltpu.sync_copy(acc.at[pl.ds(rbase, RPT)], p0.at[pl.ds(rbase, RPT)])

        @pl.when(s == NS - 1)
        def _():
            pltpu.sync_copy(acc.at[pl.ds(NS * RPT, rem)],
                            p0.at[pl.ds(NS * RPT, rem)])

    @pl.when(c == 1)
    def _():
        pltpu.sync_copy(acc.at[pl.ds(rbase, RPT)], p1.at[pl.ds(rbase, RPT)])

        @pl.when(s == NS - 1)
        def _():
            pltpu.sync_copy(acc.at[pl.ds(NS * RPT, rem)],
                            p1.at[pl.ds(NS * RPT, rem)])


_sc_aggregate = functools.partial(
    pl.kernel,
    out_type=(jax.ShapeDtypeStruct((N, D), jnp.float32),
              jax.ShapeDtypeStruct((N, D), jnp.float32)),
    mesh=plsc.VectorSubcoreMesh(core_axis_name="c", subcore_axis_name="s"),
    scratch_types=[
        pltpu.VMEM_SHARED((N, D), jnp.float32),
        pltpu.VMEM((NBE, K), jnp.int32),
        pltpu.VMEM((NBE, K), jnp.int32),
        pltpu.VMEM((NBE, K), jnp.float32),
        pltpu.VMEM((NBR, K, D), jnp.float32),
        pltpu.SemaphoreType.DMA,
        pltpu.SemaphoreType.DMA,
        pltpu.SemaphoreType.DMA,
    ],
)(_sc_body)


def _mm_body(p0_ref, p1_ref, w_ref, o_ref):
    agg = p0_ref[...] + p1_ref[...]
    acc = jnp.dot(agg, w_ref[...], preferred_element_type=jnp.float32)
    o_ref[...] = jnp.maximum(acc, 0.0)


def _matmul_relu(p0, p1, weight):
    grid = 10
    rb = N // grid
    return pl.pallas_call(
        _mm_body,
        grid=(grid,),
        in_specs=[
            pl.BlockSpec((rb, D), lambda i: (i, 0)),
            pl.BlockSpec((rb, D), lambda i: (i, 0)),
            pl.BlockSpec((D, D), lambda i: (0, 0)),
        ],
        out_specs=pl.BlockSpec((rb, D), lambda i: (i, 0)),
        out_shape=jax.ShapeDtypeStruct((N, D), jnp.float32),
    )(p0, p1, weight)


@jax.jit
def kernel(h, edge_index, edge_weight, weight):
    src1 = edge_index[0].astype(jnp.int32)
    dst1 = edge_index[1].astype(jnp.int32)
    p0, p1 = _sc_aggregate(h, src1, dst1, edge_weight)
    return _matmul_relu(p0, p1, weight)
